# Initial kernel scaffold; baseline (speedup 1.0000x reference)
import jax
import jax.numpy as jnp
from jax.experimental import pallas as pl


def kernel(pos, centroids):
    raise NotImplementedError("write your pallas kernel here")


# speedup vs baseline: 6.7615x; 6.7615x over previous
import functools

import jax
import jax.numpy as jnp
import numpy as np
from jax.experimental import pallas as pl

RSQ = np.float32(0.2 ** 2)
NNB = 64


def _tc_kernel(posT_r, cen_r, out_r, *, N, SBLK, CH):
    p3n = posT_r[0]
    cen = cen_r[0]
    cp = jax.lax.dot_general(cen, p3n, (((1,), (0,)), ((), ())),
                             preferred_element_type=jnp.float32)
    cn = cen[:, 0:1] * cen[:, 0:1] + cen[:, 1:2] * cen[:, 1:2] + cen[:, 2:3] * cen[:, 2:3]
    pn = p3n[0:1] * p3n[0:1] + p3n[1:2] * p3n[1:2] + p3n[2:3] * p3n[2:3]
    dist = -2.0 * cp
    dist = dist + cn
    dist = dist + pn
    maskf = (dist <= RSQ).astype(jnp.bfloat16)

    iu = jax.lax.broadcasted_iota(jnp.int32, (CH, CH), 0)
    it = jax.lax.broadcasted_iota(jnp.int32, (CH, CH), 1)
    tri = (iu <= it).astype(jnp.bfloat16)

    carry = jnp.zeros((SBLK, 1), jnp.float32)
    segs = []
    for k in range(N // CH):
        seg = maskf[:, k * CH:(k + 1) * CH]
        segc = jax.lax.dot_general(seg, tri, (((1,), (0,)), ((), ())),
                                   preferred_element_type=jnp.float32) + carry
        carry = segc[:, CH - 1:CH]
        segs.append(segc)
    C = jnp.concatenate(segs, axis=1)

    D = jnp.minimum(C, np.float32(NNB))
    cols = []
    for j in range(NNB):
        cols.append(jnp.sum((D <= np.float32(j)).astype(jnp.float32),
                            axis=1, keepdims=True))
    out = jnp.concatenate(cols, axis=1)
    first = out[:, 0:1]
    out = jnp.where(out >= np.float32(N), jnp.broadcast_to(first, out.shape), out)
    out_r[0] = out.astype(jnp.int32)


def kernel(pos, centroids):
    B, N, _ = pos.shape
    S = centroids.shape[1]
    SBLK = 256
    CH = 512

    posT = jnp.transpose(pos, (0, 2, 1))
    center = jnp.take_along_axis(pos, centroids[..., None].astype(jnp.int32),
                                 axis=1)

    grid = (B, S // SBLK)
    out = pl.pallas_call(
        functools.partial(_tc_kernel, N=N, SBLK=SBLK, CH=CH),
        grid=grid,
        in_specs=[
            pl.BlockSpec((1, 3, N), lambda b, s: (b, 0, 0)),
            pl.BlockSpec((1, SBLK, 3), lambda b, s: (b, s, 0)),
        ],
        out_specs=pl.BlockSpec((1, SBLK, NNB), lambda b, s: (b, s, 0)),
        out_shape=jax.ShapeDtypeStruct((B, S, NNB), jnp.int32),
    )(posT, center)
    return out


# speedup vs baseline: 20.6198x; 3.0496x over previous
import functools

import jax
import jax.numpy as jnp
import numpy as np
from jax import lax
from jax.experimental import pallas as pl
from jax.experimental.pallas import tpu as pltpu
from jax.experimental.pallas import tpu_sc as plsc

RSQ = np.float32(0.2 ** 2)
NNB = 64
NC = 2
NS = 16
NW = NC * NS


def _sc_gather_body(posf, centf, c4, posb_v, cidx_v, out_v, *, N, S, B):
    rows = (B * S) // NW
    wid = lax.axis_index("s") * NC + lax.axis_index("c")
    b = wid // (S // rows)
    base = wid * rows
    pltpu.sync_copy(posf.at[pl.ds(b * N * 3, N * 3)], posb_v)
    pltpu.sync_copy(centf.at[pl.ds(base, rows)], cidx_v)
    iota16 = lax.iota(jnp.int32, 16)
    for g in range(rows // 16):
        idx16 = cidx_v[pl.ds(g * 16, 16)] * 3
        l4 = (iota16 + g * 16) * 4
        for d in range(3):
            v = plsc.load_gather(posb_v, [idx16 + d])
            plsc.store_scatter(out_v, [l4 + d], v)
    pltpu.sync_copy(out_v, c4.at[pl.ds(base * 4, rows * 4)])


def _sc_gather(pos, centroids):
    B, N, _ = pos.shape
    S = centroids.shape[1]
    rows = (B * S) // NW
    posf = pos.reshape(B * N * 3)
    centf = centroids.reshape(B * S)
    mesh = plsc.VectorSubcoreMesh(core_axis_name="c", subcore_axis_name="s")
    k = functools.partial(
        pl.kernel,
        mesh=mesh,
        out_type=jax.ShapeDtypeStruct((B * S * 4,), jnp.float32),
        scratch_types=[
            pltpu.VMEM((N * 3,), jnp.float32),
            pltpu.VMEM((rows,), jnp.int32),
            pltpu.VMEM((rows * 4,), jnp.float32),
        ],
        compiler_params=pltpu.CompilerParams(needs_layout_passes=False),
    )(functools.partial(_sc_gather_body, N=N, S=S, B=B))
    return k(posf, centf).reshape(B, S, 4)


def _tc_body(posT_r, cen_r, c_r, *, N, SBLK, CH, NSB):
    p3n = posT_r[0]
    cen = cen_r[0][:, 0:3]
    cp = lax.dot_general(cen, p3n, (((1,), (0,)), ((), ())),
                         preferred_element_type=jnp.float32)
    cn = cen[:, 0:1] * cen[:, 0:1] + cen[:, 1:2] * cen[:, 1:2] + cen[:, 2:3] * cen[:, 2:3]
    pn = p3n[0:1] * p3n[0:1] + p3n[1:2] * p3n[1:2] + p3n[2:3] * p3n[2:3]
    dist = -2.0 * cp
    dist = dist + cn
    dist = dist + pn
    maskf = (dist <= RSQ).astype(jnp.bfloat16)

    iu = lax.broadcasted_iota(jnp.int32, (CH, CH), 0)
    it = lax.broadcasted_iota(jnp.int32, (CH, CH), 1)
    tri = (iu <= it).astype(jnp.bfloat16)

    carry = jnp.zeros((SBLK, 1), jnp.float32)
    for k in range(N // CH):
        seg = maskf[:, k * CH:(k + 1) * CH]
        segc = lax.dot_general(seg, tri, (((1,), (0,)), ((), ())),
                               preferred_element_type=jnp.float32) + carry
        carry = segc[:, CH - 1:CH]
        c_r[:, k * CH:(k + 1) * CH] = segc


def _tc_cumsum(posT, center4):
    B, _, N = posT.shape
    S = center4.shape[1]
    SBLK = 256
    CH = 512
    NSB = S // SBLK
    return pl.pallas_call(
        functools.partial(_tc_body, N=N, SBLK=SBLK, CH=CH, NSB=NSB),
        grid=(B, NSB),
        in_specs=[
            pl.BlockSpec((1, 3, N), lambda b, s: (b, 0, 0)),
            pl.BlockSpec((1, SBLK, 4), lambda b, s: (b, s, 0)),
        ],
        out_specs=pl.BlockSpec((SBLK, N), lambda b, s: (b * NSB + s, 0)),
        out_shape=jax.ShapeDtypeStruct((B * S, N), jnp.float32),
    )(posT, center4)


def _sc_extract_body(c_hbm, outf, buf0, buf1, outbuf, sem0, sem1, *, N, ROWS, CB):
    wid = lax.axis_index("s") * NC + lax.axis_index("c")
    base_row = wid * ROWS
    nchunks = ROWS // CB
    iota16 = lax.iota(jnp.int32, 16)
    jfs = [(iota16 + 16 * t).astype(jnp.float32) for t in range(NNB // 16)]
    nf = jnp.int32(N)

    pltpu.async_copy(c_hbm.at[pl.ds(base_row * N, CB * N)], buf0, sem0)

    def row_body(buf, c, r):
        rr = c * CB + r
        rN = r * N
        total = plsc.load_gather(buf, [jnp.full((16,), rN + (N - 1), jnp.int32)])
        ps = []
        for t in range(NNB // 16):
            p = jnp.zeros((16,), jnp.int32)
            step = N // 2
            while step >= 1:
                g = plsc.load_gather(buf, [p + (rN + (step - 1))])
                p = p + jnp.where(g <= jfs[t], jnp.int32(step), jnp.int32(0))
                step //= 2
            ps.append(p)
        first = jnp.broadcast_to(jnp.min(ps[0]), (16,))
        for t in range(NNB // 16):
            outv = jnp.where(jfs[t] >= total, first, ps[t])
            outbuf[pl.ds(rr * NNB + 16 * t, 16)] = outv

    def chunk_body(c, _):
        def stage(cur, nxt, cur_sem, nxt_sem):
            @pl.when(c + 1 < nchunks)
            def _():
                pltpu.async_copy(
                    c_hbm.at[pl.ds((base_row + (c + 1) * CB) * N, CB * N)],
                    nxt, nxt_sem)
            pltpu.make_async_copy(
                c_hbm.at[pl.ds((base_row + c * CB) * N, CB * N)],
                cur, cur_sem).wait()
            lax.fori_loop(0, CB, lambda r, _: row_body(cur, c, r), None)

        @pl.when(c % 2 == 0)
        def _():
            stage(buf0, buf1, sem0, sem1)

        @pl.when(c % 2 == 1)
        def _():
            stage(buf1, buf0, sem1, sem0)
        return 0

    lax.fori_loop(0, nchunks, chunk_body, 0)
    pltpu.sync_copy(outbuf, outf.at[pl.ds(base_row * NNB, ROWS * NNB)])


def _sc_extract(C, B, S, N):
    ROWS = (B * S) // NW
    CB = 8
    mesh = plsc.VectorSubcoreMesh(core_axis_name="c", subcore_axis_name="s")
    k = functools.partial(
        pl.kernel,
        mesh=mesh,
        out_type=jax.ShapeDtypeStruct((B * S * NNB,), jnp.int32),
        scratch_types=[
            pltpu.VMEM((CB * N,), jnp.float32),
            pltpu.VMEM((CB * N,), jnp.float32),
            pltpu.VMEM((ROWS * NNB,), jnp.int32),
            pltpu.SemaphoreType.DMA,
            pltpu.SemaphoreType.DMA,
        ],
        compiler_params=pltpu.CompilerParams(needs_layout_passes=False),
    )(functools.partial(_sc_extract_body, N=N, ROWS=ROWS, CB=CB))
    return k(C.reshape(B * S * N)).reshape(B, S, NNB)


def kernel(pos, centroids):
    B, N, _ = pos.shape
    S = centroids.shape[1]
    posT = jnp.transpose(pos, (0, 2, 1))
    center4 = _sc_gather(pos, centroids)
    C = _tc_cumsum(posT, center4)
    return _sc_extract(C, B, S, N)


# speedup vs baseline: 29.6757x; 1.4392x over previous
import functools

import jax
import jax.numpy as jnp
import numpy as np
from jax import lax
from jax.experimental import pallas as pl
from jax.experimental.pallas import tpu as pltpu
from jax.experimental.pallas import tpu_sc as plsc

RSQ = np.float32(0.2 ** 2)
NNB = 64
NC = 2
NS = 16
NW = NC * NS


def _sc_gather_body(posf, centf, c4, posb_v, cidx_v, out_v, *, N, S, B):
    rows = (B * S) // NW
    wid = lax.axis_index("s") * NC + lax.axis_index("c")
    b = wid // (S // rows)
    base = wid * rows
    pltpu.sync_copy(posf.at[pl.ds(b * N * 3, N * 3)], posb_v)
    pltpu.sync_copy(centf.at[pl.ds(base, rows)], cidx_v)
    iota16 = lax.iota(jnp.int32, 16)
    for g in range(rows // 16):
        idx16 = cidx_v[pl.ds(g * 16, 16)] * 3
        l4 = (iota16 + g * 16) * 4
        for d in range(3):
            v = plsc.load_gather(posb_v, [idx16 + d])
            plsc.store_scatter(out_v, [l4 + d], v)
    pltpu.sync_copy(out_v, c4.at[pl.ds(base * 4, rows * 4)])


def _sc_gather(pos, centroids):
    B, N, _ = pos.shape
    S = centroids.shape[1]
    rows = (B * S) // NW
    posf = pos.reshape(B * N * 3)
    centf = centroids.reshape(B * S)
    mesh = plsc.VectorSubcoreMesh(core_axis_name="c", subcore_axis_name="s")
    k = functools.partial(
        pl.kernel,
        mesh=mesh,
        out_type=jax.ShapeDtypeStruct((B * S * 4,), jnp.float32),
        scratch_types=[
            pltpu.VMEM((N * 3,), jnp.float32),
            pltpu.VMEM((rows,), jnp.int32),
            pltpu.VMEM((rows * 4,), jnp.float32),
        ],
        compiler_params=pltpu.CompilerParams(needs_layout_passes=False),
    )(functools.partial(_sc_gather_body, N=N, S=S, B=B))
    return k(posf, centf).reshape(B, S, 4)


def _tc_body(posT_r, cen_r, c_r, *, N, SBLK, CH, NSB):
    p3n = posT_r[0]
    cen = cen_r[0][:, 0:3]
    cp = lax.dot_general(cen, p3n, (((1,), (0,)), ((), ())),
                         preferred_element_type=jnp.float32)
    cn = cen[:, 0:1] * cen[:, 0:1] + cen[:, 1:2] * cen[:, 1:2] + cen[:, 2:3] * cen[:, 2:3]
    pn = p3n[0:1] * p3n[0:1] + p3n[1:2] * p3n[1:2] + p3n[2:3] * p3n[2:3]
    dist = -2.0 * cp
    dist = dist + cn
    dist = dist + pn
    maskf = (dist <= RSQ).astype(jnp.bfloat16)

    iu = lax.broadcasted_iota(jnp.int32, (CH, CH), 0)
    it = lax.broadcasted_iota(jnp.int32, (CH, CH), 1)
    tri = (iu <= it).astype(jnp.bfloat16)

    carry = jnp.zeros((SBLK, 1), jnp.float32)
    for k in range(N // CH):
        seg = maskf[:, k * CH:(k + 1) * CH]
        segc = lax.dot_general(seg, tri, (((1,), (0,)), ((), ())),
                               preferred_element_type=jnp.float32) + carry
        carry = segc[:, CH - 1:CH]
        c_r[:, k * CH:(k + 1) * CH] = segc


def _tc_cumsum(posT, center4):
    B, _, N = posT.shape
    S = center4.shape[1]
    SBLK = 256
    CH = 512
    NSB = S // SBLK
    return pl.pallas_call(
        functools.partial(_tc_body, N=N, SBLK=SBLK, CH=CH, NSB=NSB),
        grid=(B, NSB),
        in_specs=[
            pl.BlockSpec((1, 3, N), lambda b, s: (b, 0, 0)),
            pl.BlockSpec((1, SBLK, 4), lambda b, s: (b, s, 0)),
        ],
        out_specs=pl.BlockSpec((SBLK, N), lambda b, s: (b * NSB + s, 0)),
        out_shape=jax.ShapeDtypeStruct((B * S, N), jnp.float32),
    )(posT, center4)


def _sc_extract_body(c_hbm, outf, buf0, buf1, outbuf, sem0, sem1, *, N, ROWS, CB):
    wid = lax.axis_index("s") * NC + lax.axis_index("c")
    base_row = wid * ROWS
    nchunks = ROWS // CB
    iota16 = lax.iota(jnp.int32, 16)
    jfs = [(iota16 + 16 * t).astype(jnp.float32) for t in range(NNB // 16)]
    nf = jnp.int32(N)

    pltpu.async_copy(c_hbm.at[pl.ds(base_row, CB)], buf0, sem0)

    def row_body(buf, c, r):
        rr = c * CB + r
        rsplat = jnp.full((16,), r, jnp.int32)
        total = plsc.load_gather(buf, [rsplat, jnp.full((16,), N - 1, jnp.int32)])
        ps = []
        for t in range(NNB // 16):
            p = jnp.zeros((16,), jnp.int32)
            step = N // 2
            while step >= 1:
                g = plsc.load_gather(buf, [rsplat, p + (step - 1)])
                p = p + jnp.where(g <= jfs[t], jnp.int32(step), jnp.int32(0))
                step //= 2
            ps.append(p)
        first = jnp.broadcast_to(jnp.min(ps[0]), (16,))
        for t in range(NNB // 16):
            outv = jnp.where(jfs[t] >= total, first, ps[t])
            outbuf[pl.ds(rr * NNB + 16 * t, 16)] = outv

    def chunk_body(c, _):
        def stage(cur, nxt, cur_sem, nxt_sem):
            @pl.when(c + 1 < nchunks)
            def _():
                pltpu.async_copy(
                    c_hbm.at[pl.ds(base_row + (c + 1) * CB, CB)], nxt, nxt_sem)
            pltpu.make_async_copy(
                c_hbm.at[pl.ds(base_row + c * CB, CB)], cur, cur_sem).wait()
            lax.fori_loop(0, CB, lambda r, _: row_body(cur, c, r), None)

        @pl.when(c % 2 == 0)
        def _():
            stage(buf0, buf1, sem0, sem1)

        @pl.when(c % 2 == 1)
        def _():
            stage(buf1, buf0, sem1, sem0)
        return 0

    lax.fori_loop(0, nchunks, chunk_body, 0)
    pltpu.sync_copy(outbuf, outf.at[pl.ds(base_row * NNB, ROWS * NNB)])


def _sc_extract(C, B, S, N):
    ROWS = (B * S) // NW
    CB = 8
    mesh = plsc.VectorSubcoreMesh(core_axis_name="c", subcore_axis_name="s")
    k = functools.partial(
        pl.kernel,
        mesh=mesh,
        out_type=jax.ShapeDtypeStruct((B * S * NNB,), jnp.int32),
        scratch_types=[
            pltpu.VMEM((CB, N), jnp.float32),
            pltpu.VMEM((CB, N), jnp.float32),
            pltpu.VMEM((ROWS * NNB,), jnp.int32),
            pltpu.SemaphoreType.DMA,
            pltpu.SemaphoreType.DMA,
        ],
        compiler_params=pltpu.CompilerParams(needs_layout_passes=False),
    )(functools.partial(_sc_extract_body, N=N, ROWS=ROWS, CB=CB))
    return k(C).reshape(B, S, NNB)


def kernel(pos, centroids):
    B, N, _ = pos.shape
    S = centroids.shape[1]
    posT = jnp.transpose(pos, (0, 2, 1))
    center4 = _sc_gather(pos, centroids)
    C = _tc_cumsum(posT, center4)
    return _sc_extract(C, B, S, N)


# speedup vs baseline: 31.0668x; 1.0469x over previous
import functools

import jax
import jax.numpy as jnp
import numpy as np
from jax import lax
from jax.experimental import pallas as pl
from jax.experimental.pallas import tpu as pltpu
from jax.experimental.pallas import tpu_sc as plsc

RSQ = np.float32(0.2 ** 2)
NNB = 64
NC = 2
NS = 16
NW = NC * NS


def _sc_gather_body(posf, centf, c4, posb_v, cidx_v, out_v, *, N, S, B):
    rows = (B * S) // NW
    wid = lax.axis_index("s") * NC + lax.axis_index("c")
    b = wid // (S // rows)
    base = wid * rows
    pltpu.sync_copy(posf.at[pl.ds(b * N * 3, N * 3)], posb_v)
    pltpu.sync_copy(centf.at[pl.ds(base, rows)], cidx_v)
    iota16 = lax.iota(jnp.int32, 16)
    for g in range(rows // 16):
        idx16 = cidx_v[pl.ds(g * 16, 16)] * 3
        l4 = (iota16 + g * 16) * 4
        for d in range(3):
            v = plsc.load_gather(posb_v, [idx16 + d])
            plsc.store_scatter(out_v, [l4 + d], v)
    pltpu.sync_copy(out_v, c4.at[pl.ds(base * 4, rows * 4)])


def _sc_gather(pos, centroids):
    B, N, _ = pos.shape
    S = centroids.shape[1]
    rows = (B * S) // NW
    posf = pos.reshape(B * N * 3)
    centf = centroids.reshape(B * S)
    mesh = plsc.VectorSubcoreMesh(core_axis_name="c", subcore_axis_name="s")
    k = functools.partial(
        pl.kernel,
        mesh=mesh,
        out_type=jax.ShapeDtypeStruct((B * S * 4,), jnp.float32),
        scratch_types=[
            pltpu.VMEM((N * 3,), jnp.float32),
            pltpu.VMEM((rows,), jnp.int32),
            pltpu.VMEM((rows * 4,), jnp.float32),
        ],
        compiler_params=pltpu.CompilerParams(needs_layout_passes=False),
    )(functools.partial(_sc_gather_body, N=N, S=S, B=B))
    return k(posf, centf).reshape(B, S, 4)


def _tc_body(posT_r, cen_r, o_r, m_r, *, N, SBLK, W):
    p3n = posT_r[0]
    cen = cen_r[0][:, 0:3]
    cp = lax.dot_general(cen, p3n, (((1,), (0,)), ((), ())),
                         preferred_element_type=jnp.float32)
    cn = cen[:, 0:1] * cen[:, 0:1] + cen[:, 1:2] * cen[:, 1:2] + cen[:, 2:3] * cen[:, 2:3]
    pn = p3n[0:1] * p3n[0:1] + p3n[1:2] * p3n[1:2] + p3n[2:3] * p3n[2:3]
    dist = -2.0 * cp
    dist = dist + cn
    dist = dist + pn
    maskf = (dist <= RSQ).astype(jnp.bfloat16)

    ii = lax.broadcasted_iota(jnp.int32, (N, W), 0)
    iw = lax.broadcasted_iota(jnp.int32, (N, W), 1)
    sel = (ii >> 4) == iw
    bd = sel.astype(jnp.bfloat16)
    pw = jnp.where(sel, jnp.int32(1) << (ii & 15),
                   jnp.int32(0)).astype(jnp.bfloat16)
    counts = lax.dot_general(maskf, bd, (((1,), (0,)), ((), ())),
                             preferred_element_type=jnp.float32)
    mwords = lax.dot_general(maskf, pw, (((1,), (0,)), ((), ())),
                             preferred_element_type=jnp.float32)
    iu = lax.broadcasted_iota(jnp.int32, (W, W), 0)
    it = lax.broadcasted_iota(jnp.int32, (W, W), 1)
    tri = (iu <= it).astype(jnp.bfloat16)
    off = lax.dot_general(counts.astype(jnp.bfloat16), tri,
                          (((1,), (0,)), ((), ())),
                          preferred_element_type=jnp.float32)
    o_r[...] = off.astype(jnp.int32)
    m_r[...] = mwords.astype(jnp.int32)


def _tc_stats(posT, center4):
    B, _, N = posT.shape
    S = center4.shape[1]
    SBLK = 256
    W = N // 16
    NSB = S // SBLK
    return pl.pallas_call(
        functools.partial(_tc_body, N=N, SBLK=SBLK, W=W),
        grid=(B, NSB),
        in_specs=[
            pl.BlockSpec((1, 3, N), lambda b, s: (b, 0, 0)),
            pl.BlockSpec((1, SBLK, 4), lambda b, s: (b, s, 0)),
        ],
        out_specs=[
            pl.BlockSpec((SBLK, W), lambda b, s: (b * NSB + s, 0)),
            pl.BlockSpec((SBLK, W), lambda b, s: (b * NSB + s, 0)),
        ],
        out_shape=[
            jax.ShapeDtypeStruct((B * S, W), jnp.int32),
            jax.ShapeDtypeStruct((B * S, W), jnp.int32),
        ],
    )(posT, center4)


def _popcount16(x):
    x = x - ((x >> 1) & 0x5555)
    x = (x & 0x3333) + ((x >> 2) & 0x3333)
    x = (x + (x >> 4)) & 0x0F0F
    return (x + (x >> 8)) & 0x1F


def _sc_extract_body(o_hbm, m_hbm, outf, ob0, ob1, mb0, mb1, outbuf, sem0, sem1,
                     *, W, ROWS, CB):
    wid = lax.axis_index("s") * NC + lax.axis_index("c")
    base_row = wid * ROWS
    nchunks = ROWS // CB
    iota16 = lax.iota(jnp.int32, 16)
    jis = [iota16 + 16 * t for t in range(NNB // 16)]

    def start(c, ob, mb, sem):
        pltpu.async_copy(o_hbm.at[pl.ds(base_row + c * CB, CB)], ob, sem)
        pltpu.async_copy(m_hbm.at[pl.ds(base_row + c * CB, CB)], mb, sem)

    def waitc(c, ob, mb, sem):
        pltpu.make_async_copy(
            o_hbm.at[pl.ds(base_row + c * CB, CB)], ob, sem).wait()
        pltpu.make_async_copy(
            m_hbm.at[pl.ds(base_row + c * CB, CB)], mb, sem).wait()

    start(0, ob0, mb0, sem0)

    def row_body(ob, mb, c, r):
        rr = c * CB + r
        rsplat = jnp.full((16,), r, jnp.int32)
        total = plsc.load_gather(ob, [rsplat, jnp.full((16,), W - 1, jnp.int32)])
        ps = []
        for t in range(NNB // 16):
            j = jis[t]
            k = jnp.zeros((16,), jnp.int32)
            step = W // 2
            while step >= 1:
                g = plsc.load_gather(ob, [rsplat, k + (step - 1)])
                k = k + jnp.where(g <= j, jnp.int32(step), jnp.int32(0))
                step //= 2
            prev = plsc.load_gather(ob, [rsplat, jnp.maximum(k - 1, 0)])
            m = j - jnp.where(k == 0, jnp.int32(0), prev)
            w = plsc.load_gather(mb, [rsplat, k])
            q = jnp.zeros((16,), jnp.int32)
            s2 = 8
            while s2 >= 1:
                pref = w & ((jnp.int32(2) << (q + (s2 - 1))) - 1)
                a = _popcount16(pref)
                q = q + jnp.where(a <= m, jnp.int32(s2), jnp.int32(0))
                s2 //= 2
            ps.append(k * 16 + q)
        first = jnp.broadcast_to(jnp.min(ps[0]), (16,))
        for t in range(NNB // 16):
            outv = jnp.where(jis[t] >= total, first, ps[t])
            outbuf[pl.ds(rr * NNB + 16 * t, 16)] = outv

    def chunk_body(c, _):
        def stage(cur_o, cur_m, nxt_o, nxt_m, cur_sem, nxt_sem):
            @pl.when(c + 1 < nchunks)
            def _():
                start(c + 1, nxt_o, nxt_m, nxt_sem)
            waitc(c, cur_o, cur_m, cur_sem)
            lax.fori_loop(0, CB, lambda r, _: row_body(cur_o, cur_m, c, r), None)

        @pl.when(c % 2 == 0)
        def _():
            stage(ob0, mb0, ob1, mb1, sem0, sem1)

        @pl.when(c % 2 == 1)
        def _():
            stage(ob1, mb1, ob0, mb0, sem1, sem0)
        return 0

    lax.fori_loop(0, nchunks, chunk_body, 0)
    pltpu.sync_copy(outbuf, outf.at[pl.ds(base_row * NNB, ROWS * NNB)])


def _sc_extract(O, M, B, S, N):
    W = N // 16
    ROWS = (B * S) // NW
    CB = 64
    mesh = plsc.VectorSubcoreMesh(core_axis_name="c", subcore_axis_name="s")
    k = functools.partial(
        pl.kernel,
        mesh=mesh,
        out_type=jax.ShapeDtypeStruct((B * S * NNB,), jnp.int32),
        scratch_types=[
            pltpu.VMEM((CB, W), jnp.int32),
            pltpu.VMEM((CB, W), jnp.int32),
            pltpu.VMEM((CB, W), jnp.int32),
            pltpu.VMEM((CB, W), jnp.int32),
            pltpu.VMEM((ROWS * NNB,), jnp.int32),
            pltpu.SemaphoreType.DMA,
            pltpu.SemaphoreType.DMA,
        ],
        compiler_params=pltpu.CompilerParams(needs_layout_passes=False),
    )(functools.partial(_sc_extract_body, W=W, ROWS=ROWS, CB=CB))
    return k(O, M).reshape(B, S, NNB)


def kernel(pos, centroids):
    B, N, _ = pos.shape
    S = centroids.shape[1]
    posT = jnp.transpose(pos, (0, 2, 1))
    center4 = _sc_gather(pos, centroids)
    O, M = _tc_stats(posT, center4)
    return _sc_extract(O, M, B, S, N)


# speedup vs baseline: 32.1560x; 1.0351x over previous
import functools

import jax
import jax.numpy as jnp
import numpy as np
from jax import lax
from jax.experimental import pallas as pl
from jax.experimental.pallas import tpu as pltpu
from jax.experimental.pallas import tpu_sc as plsc

RSQ = np.float32(0.2 ** 2)
NNB = 64
NC = 2
NS = 16
NW = NC * NS


def _sc_gather_body(posf, centf, c4, posb_v, cidx_v, out_v, *, N, S, B):
    rows = (B * S) // NW
    wid = lax.axis_index("s") * NC + lax.axis_index("c")
    b = wid // (S // rows)
    base = wid * rows
    pltpu.sync_copy(posf.at[pl.ds(b * N * 3, N * 3)], posb_v)
    pltpu.sync_copy(centf.at[pl.ds(base, rows)], cidx_v)
    iota16 = lax.iota(jnp.int32, 16)
    for g in range(rows // 16):
        idx16 = cidx_v[pl.ds(g * 16, 16)] * 3
        l4 = (iota16 + g * 16) * 4
        for d in range(3):
            v = plsc.load_gather(posb_v, [idx16 + d])
            plsc.store_scatter(out_v, [l4 + d], v)
    pltpu.sync_copy(out_v, c4.at[pl.ds(base * 4, rows * 4)])


def _sc_gather(pos, centroids):
    B, N, _ = pos.shape
    S = centroids.shape[1]
    rows = (B * S) // NW
    posf = pos.reshape(B * N * 3)
    centf = centroids.reshape(B * S)
    mesh = plsc.VectorSubcoreMesh(core_axis_name="c", subcore_axis_name="s")
    k = functools.partial(
        pl.kernel,
        mesh=mesh,
        out_type=jax.ShapeDtypeStruct((B * S * 4,), jnp.float32),
        scratch_types=[
            pltpu.VMEM((N * 3,), jnp.float32),
            pltpu.VMEM((rows,), jnp.int32),
            pltpu.VMEM((rows * 4,), jnp.float32),
        ],
        compiler_params=pltpu.CompilerParams(needs_layout_passes=False),
    )(functools.partial(_sc_gather_body, N=N, S=S, B=B))
    return k(posf, centf).reshape(B, S, 4)


def _tc_body(posT_r, cen_r, bp_r, tri_r, o_r, m_r, *, N, SBLK, W):
    p3n = posT_r[0]
    cen = cen_r[0][:, 0:3]
    cp = lax.dot_general(cen, p3n, (((1,), (0,)), ((), ())),
                         preferred_element_type=jnp.float32)
    cn = cen[:, 0:1] * cen[:, 0:1] + cen[:, 1:2] * cen[:, 1:2] + cen[:, 2:3] * cen[:, 2:3]
    pn = p3n[0:1] * p3n[0:1] + p3n[1:2] * p3n[1:2] + p3n[2:3] * p3n[2:3]
    dist = -2.0 * cp
    dist = dist + cn
    dist = dist + pn
    maskf = (dist <= RSQ).astype(jnp.bfloat16)

    cm = lax.dot_general(maskf, bp_r[...], (((1,), (0,)), ((), ())),
                         preferred_element_type=jnp.float32)
    counts = cm[:, 0:W]
    mwords = cm[:, W:2 * W]
    off = lax.dot_general(counts.astype(jnp.bfloat16), tri_r[...],
                          (((1,), (0,)), ((), ())),
                          preferred_element_type=jnp.float32)
    o_r[...] = off.astype(jnp.int32)
    m_r[...] = mwords.astype(jnp.int32)


def _tc_stats(posT, center4):
    B, _, N = posT.shape
    S = center4.shape[1]
    SBLK = 256
    W = N // 16
    NSB = S // SBLK

    ii = lax.broadcasted_iota(jnp.int32, (N, W), 0)
    iw = lax.broadcasted_iota(jnp.int32, (N, W), 1)
    sel = (ii >> 4) == iw
    bd = sel.astype(jnp.bfloat16)
    pw = jnp.where(sel, jnp.int32(1) << (ii & 15),
                   jnp.int32(0)).astype(jnp.bfloat16)
    bp = jnp.concatenate([bd, pw], axis=1)
    iu = lax.broadcasted_iota(jnp.int32, (W, W), 0)
    it = lax.broadcasted_iota(jnp.int32, (W, W), 1)
    tri = (iu <= it).astype(jnp.bfloat16)

    return pl.pallas_call(
        functools.partial(_tc_body, N=N, SBLK=SBLK, W=W),
        grid=(B, NSB),
        in_specs=[
            pl.BlockSpec((1, 3, N), lambda b, s: (b, 0, 0)),
            pl.BlockSpec((1, SBLK, 4), lambda b, s: (b, s, 0)),
            pl.BlockSpec((N, 2 * W), lambda b, s: (0, 0)),
            pl.BlockSpec((W, W), lambda b, s: (0, 0)),
        ],
        out_specs=[
            pl.BlockSpec((SBLK, W), lambda b, s: (b * NSB + s, 0)),
            pl.BlockSpec((SBLK, W), lambda b, s: (b * NSB + s, 0)),
        ],
        out_shape=[
            jax.ShapeDtypeStruct((B * S, W), jnp.int32),
            jax.ShapeDtypeStruct((B * S, W), jnp.int32),
        ],
    )(posT, center4, bp, tri)


def _popcount16(x):
    x = x - ((x >> 1) & 0x5555)
    x = (x & 0x3333) + ((x >> 2) & 0x3333)
    x = (x + (x >> 4)) & 0x0F0F
    return (x + (x >> 8)) & 0x1F


def _sc_extract_body(o_hbm, m_hbm, outf, ob0, ob1, mb0, mb1, outbuf, sem0, sem1,
                     *, W, ROWS, CB):
    wid = lax.axis_index("s") * NC + lax.axis_index("c")
    base_row = wid * ROWS
    nchunks = ROWS // CB
    iota16 = lax.iota(jnp.int32, 16)
    jis = [iota16 + 16 * t for t in range(NNB // 16)]

    def start(c, ob, mb, sem):
        pltpu.async_copy(o_hbm.at[pl.ds(base_row + c * CB, CB)], ob, sem)
        pltpu.async_copy(m_hbm.at[pl.ds(base_row + c * CB, CB)], mb, sem)

    def waitc(c, ob, mb, sem):
        pltpu.make_async_copy(
            o_hbm.at[pl.ds(base_row + c * CB, CB)], ob, sem).wait()
        pltpu.make_async_copy(
            m_hbm.at[pl.ds(base_row + c * CB, CB)], mb, sem).wait()

    start(0, ob0, mb0, sem0)

    def row_body(ob, mb, c, r):
        rr = c * CB + r
        rsplat = jnp.full((16,), r, jnp.int32)
        total = plsc.load_gather(ob, [rsplat, jnp.full((16,), W - 1, jnp.int32)])
        ps = []
        for t in range(NNB // 16):
            j = jis[t]
            k = jnp.zeros((16,), jnp.int32)
            step = W // 2
            while step >= 1:
                g = plsc.load_gather(ob, [rsplat, k + (step - 1)])
                k = k + jnp.where(g <= j, jnp.int32(step), jnp.int32(0))
                step //= 2
            prev = plsc.load_gather(ob, [rsplat, jnp.maximum(k - 1, 0)])
            m = j - jnp.where(k == 0, jnp.int32(0), prev)
            w = plsc.load_gather(mb, [rsplat, k])
            q = jnp.zeros((16,), jnp.int32)
            s2 = 8
            while s2 >= 1:
                pref = w & ((jnp.int32(2) << (q + (s2 - 1))) - 1)
                a = _popcount16(pref)
                q = q + jnp.where(a <= m, jnp.int32(s2), jnp.int32(0))
                s2 //= 2
            ps.append(k * 16 + q)
        first = jnp.broadcast_to(jnp.min(ps[0]), (16,))
        for t in range(NNB // 16):
            outv = jnp.where(jis[t] >= total, first, ps[t])
            outbuf[pl.ds(rr * NNB + 16 * t, 16)] = outv

    def chunk_body(c, _):
        def stage(cur_o, cur_m, nxt_o, nxt_m, cur_sem, nxt_sem):
            @pl.when(c + 1 < nchunks)
            def _():
                start(c + 1, nxt_o, nxt_m, nxt_sem)
            waitc(c, cur_o, cur_m, cur_sem)
            lax.fori_loop(0, CB, lambda r, _: row_body(cur_o, cur_m, c, r), None)

        @pl.when(c % 2 == 0)
        def _():
            stage(ob0, mb0, ob1, mb1, sem0, sem1)

        @pl.when(c % 2 == 1)
        def _():
            stage(ob1, mb1, ob0, mb0, sem1, sem0)
        return 0

    lax.fori_loop(0, nchunks, chunk_body, 0)
    pltpu.sync_copy(outbuf, outf.at[pl.ds(base_row * NNB, ROWS * NNB)])


def _sc_extract(O, M, B, S, N):
    W = N // 16
    ROWS = (B * S) // NW
    CB = 64
    mesh = plsc.VectorSubcoreMesh(core_axis_name="c", subcore_axis_name="s")
    k = functools.partial(
        pl.kernel,
        mesh=mesh,
        out_type=jax.ShapeDtypeStruct((B * S * NNB,), jnp.int32),
        scratch_types=[
            pltpu.VMEM((CB, W), jnp.int32),
            pltpu.VMEM((CB, W), jnp.int32),
            pltpu.VMEM((CB, W), jnp.int32),
            pltpu.VMEM((CB, W), jnp.int32),
            pltpu.VMEM((ROWS * NNB,), jnp.int32),
            pltpu.SemaphoreType.DMA,
            pltpu.SemaphoreType.DMA,
        ],
        compiler_params=pltpu.CompilerParams(needs_layout_passes=False),
    )(functools.partial(_sc_extract_body, W=W, ROWS=ROWS, CB=CB))
    return k(O, M).reshape(B, S, NNB)


def kernel(pos, centroids):
    B, N, _ = pos.shape
    S = centroids.shape[1]
    posT = jnp.transpose(pos, (0, 2, 1))
    center4 = _sc_gather(pos, centroids)
    O, M = _tc_stats(posT, center4)
    return _sc_extract(O, M, B, S, N)
